# baseline (speedup 1.0000x reference)
import functools

import jax
import jax.numpy as jnp
from jax import lax
from jax.experimental import pallas as pl
from jax.experimental.pallas import tpu as pltpu
import jax.experimental.pallas.tpu_sc as plsc

_N = 10000
_E = 320000
_B = 16
_H = 256
_NC = 2
_NS = 16
_K = 128
_NP = 10240
_RPT = _NP // _NS
_RS = 2560
_TRS = _RS // _NS
_RE = 1280
_TRE = _RE // _NS
_BR = 1000


def _zero_rows(buf, n):
    zero16 = jnp.zeros((16,), jnp.float32)

    def _z(i, _):
        for j in range(128 // 16):
            buf[i, pl.ds(j * 16, 16)] = zero16
        return _
    lax.fori_loop(0, n, _z, None)


def _zero_acc(acc, rows, s):
    r0 = s * _RPT
    for j in range(5):
        pltpu.sync_copy(rows.at[pl.ds(0, _K)],
                        acc.at[pl.ds(r0 + j * _K, _K)])


def _acc_writeout(acc, out, c, s):
    r0 = s * _RPT
    pltpu.sync_copy(acc.at[pl.ds(r0, _RPT)],
                    out.at[pl.ds(c * _NP + r0, _RPT)])


def _agg_pipeline(table, acc, srcs, dst, sbase, dbase, nchunks,
                  iss, ids, isems, rows, gsems, dhook=None):
    ssems = gsems[2:]
    gsems = gsems[:2]

    def ld(chunk, j):
        off = chunk * _K
        pltpu.async_copy(srcs.at[pl.ds(sbase + off, _K)], iss[j], isems[j])
        pltpu.async_copy(dst.at[pl.ds(dbase + off, _K)], ids[j], isems[j])

    def wld(j):
        pltpu.make_async_copy(srcs.at[pl.ds(0, _K)], iss[j], isems[j]).wait()
        pltpu.make_async_copy(dst.at[pl.ds(0, _K)], ids[j], isems[j]).wait()

    def g(j, slot):
        pltpu.async_copy(table.at[iss[j]], rows[slot], gsems[slot])

    def wg(slot):
        pltpu.make_async_copy(table.at[pl.ds(0, _K)], rows[slot],
                              gsems[slot]).wait()

    def sc(j, slot):
        pltpu.async_copy(rows[slot], acc.at[ids[j]], ssems[slot], add=True)
        if dhook is not None:
            dhook(j)

    def wsc(j, slot):
        pltpu.make_async_copy(rows[slot], acc.at[ids[j]], ssems[slot]).wait()

    ld(0, 0)
    ld(1, 1)
    ld(2, 2)
    wld(0)
    g(0, 0)

    nb = nchunks // 4

    def _body(t, carry):
        b = 4 * t
        nxt = t < nb - 1
        ld(b + 3, 3)
        wld(1)

        @pl.when(t > 0)
        def _():
            wsc(3, 1)
        g(1, 1)
        wg(0)
        sc(0, 0)

        @pl.when(nxt)
        def _():
            ld(b + 4, 0)
        wld(2)
        wsc(0, 0)
        g(2, 0)
        wg(1)
        sc(1, 1)

        @pl.when(nxt)
        def _():
            ld(b + 5, 1)
        wld(3)
        wsc(1, 1)
        g(3, 1)
        wg(0)
        sc(2, 0)

        @pl.when(nxt)
        def _():
            ld(b + 6, 2)
            wld(0)
            wsc(2, 0)
            g(0, 0)
        wg(1)
        sc(3, 1)
        return carry
    lax.fori_loop(0, nb, _body, None)
    wsc(2, 0)
    wsc(3, 1)


def _sc_agg_common(table, srcs, dst, agg, sbase, dbase, nchunks, refs):
    (acc, i0, i1, i2, i3, d0, d1, d2, d3, r0, r1,
     m0, m1, m2, m3, gs0, gs1, ss0, ss1) = refs
    c = lax.axis_index("c")
    s = lax.axis_index("s")
    _zero_rows(r0, _K)
    _zero_acc(acc, r0, s)
    plsc.subcore_barrier()
    _agg_pipeline(table, acc, srcs, dst, sbase, dbase, nchunks,
                  (i0, i1, i2, i3), (d0, d1, d2, d3), (m0, m1, m2, m3),
                  (r0, r1), (gs0, gs1, ss0, ss1))
    plsc.subcore_barrier()
    _acc_writeout(acc, agg, c, s)


def _sc_slab_body(table, srcs, dst, agg, *refs):
    c = lax.axis_index("c")
    s = lax.axis_index("s")
    base = (c * _RS + s * _TRS) * _K
    _sc_agg_common(table, srcs, dst, agg, base, (s * _TRS) * _K, _TRS, refs)


def _sc_esplit_body(table, srcs, dst, agg, deg, *refs):
    (acc, dacc, i0, i1, i2, i3, d0, d1, d2, d3, r0, r1,
     m0, m1, m2, m3, gs0, gs1, ss0, ss1, ones, dbuf) = refs
    c = lax.axis_index("c")
    s = lax.axis_index("s")
    zero16 = jnp.zeros((16,), jnp.float32)
    _zero_rows(r0, _K)
    _zero_acc(acc, r0, s)
    r0b = s * _RPT
    for j in range(8):
        ones[pl.ds(j * 16, 16)] = zero16
    for j in range(5):
        pltpu.sync_copy(ones, dacc.at[pl.ds(r0b + j * _K, _K)])
    one16 = jnp.ones((16,), jnp.float32)
    for j in range(8):
        ones[pl.ds(j * 16, 16)] = one16
    plsc.subcore_barrier()

    base = (c * _RE + s * _TRE) * _K
    ids = (d0, d1, d2, d3)

    def dhook(j):
        pltpu.sync_copy(ones, dacc.at[ids[j]], add=True)

    _agg_pipeline(table, acc, srcs, dst, base, base, _TRE,
                  (i0, i1, i2, i3), ids, (m0, m1, m2, m3),
                  (r0, r1), (gs0, gs1, ss0, ss1), dhook)
    plsc.subcore_barrier()
    _acc_writeout(acc, agg, c, s)

    pltpu.sync_copy(dacc.at[pl.ds(r0b, _RPT)], dbuf)
    for blk in range(5):
        def _brow(g, carry):
            v16 = dbuf[pl.ds(blk * _K + g * 16, 16)]
            for l in range(16):
                row = jnp.full((16,), v16[l], jnp.float32)
                for j in range(8):
                    r0[g * 16 + l, pl.ds(j * 16, 16)] = row
            return carry
        lax.fori_loop(0, 8, _brow, None)
        pltpu.sync_copy(r0, deg.at[pl.ds(c * _NP + r0b + blk * _K, _K)])


def _sc_mesh():
    return plsc.VectorSubcoreMesh(core_axis_name="c", subcore_axis_name="s",
                                  num_cores=_NC, num_subcores=_NS)


def _agg_scratch():
    return (
        (pltpu.VMEM_SHARED((_NP, 128), jnp.float32),)
        + (pltpu.VMEM((_K,), jnp.int32),) * 8
        + (pltpu.VMEM((_K, 128), jnp.float32),) * 2
        + (pltpu.SemaphoreType.DMA,) * 8
    )


_SC_OUT = jax.ShapeDtypeStruct((_NC * _NP, 128), jnp.float32)


@functools.lru_cache(maxsize=None)
def _make_sc_slab():
    return pl.kernel(_sc_slab_body, out_type=_SC_OUT,
                     mesh=_sc_mesh(), scratch_types=_agg_scratch())


@functools.lru_cache(maxsize=None)
def _make_sc_esplit():
    scratch = (
        (pltpu.VMEM_SHARED((_NP, 128), jnp.float32),
         pltpu.VMEM_SHARED((_NP,), jnp.float32))
        + (pltpu.VMEM((_K,), jnp.int32),) * 8
        + (pltpu.VMEM((_K, 128), jnp.float32),) * 2
        + (pltpu.SemaphoreType.DMA,) * 8
        + (pltpu.VMEM((_K,), jnp.float32),
           pltpu.VMEM((_RPT,), jnp.float32))
    )
    return pl.kernel(_sc_esplit_body, out_type=(_SC_OUT, _SC_OUT),
                     mesh=_sc_mesh(), scratch_types=scratch)


def _mm_body(mode, agg_ref, h_ref, deg_ref, wl_ref, bl_ref, wr_ref,
             y_ref, st_ref):
    dn = (((1,), (1,)), ((), ()))
    if mode == "sum":
        aggc = agg_ref[0] + agg_ref[1]
        hc = h_ref[...]
    else:
        aggc = jnp.concatenate([agg_ref[0], agg_ref[1]], axis=1)
        hc = jnp.concatenate([h_ref[0], h_ref[1]], axis=1)
    bf = jnp.bfloat16
    yl = lax.dot_general(aggc.astype(bf), wl_ref[...].astype(bf), dn,
                         preferred_element_type=jnp.float32)
    d = deg_ref[0, :, 0:1] + deg_ref[1, :, 0:1]
    rdeg = 1.0 / jnp.maximum(d, 1.0)
    y = (yl * rdeg + bl_ref[...]
         + lax.dot_general(hc.astype(bf), wr_ref[...].astype(bf), dn,
                           preferred_element_type=jnp.float32))
    y_ref[...] = y
    s1 = jnp.sum(y, axis=0, keepdims=True)
    s2 = jnp.sum(y * y, axis=0, keepdims=True)
    blk = jnp.concatenate([s1, s2, jnp.zeros((6, _H), jnp.float32)], axis=0)

    @pl.when(pl.program_id(0) == 0)
    def _():
        st_ref[...] = blk

    @pl.when(pl.program_id(0) != 0)
    def _():
        st_ref[...] = st_ref[...] + blk


def _mm_layer(agg, h, deg, wl, bl, wr, mode):
    nb = _N // _BR
    din = wl.shape[1]
    if mode == "sum":
        h_spec = pl.BlockSpec((_BR, 128), lambda i: (i, 0))
    else:
        h_spec = pl.BlockSpec((2, _BR, 128), lambda i: (0, i, 0))
    return pl.pallas_call(
        functools.partial(_mm_body, mode),
        grid=(nb,),
        in_specs=[
            pl.BlockSpec((2, _BR, 128), lambda i: (0, i, 0)),
            h_spec,
            pl.BlockSpec((2, _BR, 128), lambda i: (0, i, 0)),
            pl.BlockSpec((_H, din), lambda i: (0, 0)),
            pl.BlockSpec((1, _H), lambda i: (0, 0)),
            pl.BlockSpec((_H, din), lambda i: (0, 0)),
        ],
        out_specs=[
            pl.BlockSpec((_BR, _H), lambda i: (i, 0)),
            pl.BlockSpec((8, _H), lambda i: (0, 0)),
        ],
        out_shape=[
            jax.ShapeDtypeStruct((_N, _H), jnp.float32),
            jax.ShapeDtypeStruct((8, _H), jnp.float32),
        ],
    )(agg, h, deg, wl, bl, wr)


def _norm_body(y_ref, st_ref, g_ref, b_ref, out_ref):
    mean = st_ref[0:1, :] / _N
    var = st_ref[1:2, :] / _N - mean * mean
    inv = lax.rsqrt(var + 1e-5)
    h = jnp.maximum((y_ref[...] - mean) * inv * g_ref[...] + b_ref[...], 0.0)
    out_ref[0] = h[:, 0:128]
    out_ref[1] = h[:, 128:256]


def _norm_layer(y, st, g, b):
    nb = _N // _BR
    return pl.pallas_call(
        _norm_body,
        grid=(nb,),
        in_specs=[
            pl.BlockSpec((_BR, _H), lambda i: (i, 0)),
            pl.BlockSpec((8, _H), lambda i: (0, 0)),
            pl.BlockSpec((1, _H), lambda i: (0, 0)),
            pl.BlockSpec((1, _H), lambda i: (0, 0)),
        ],
        out_specs=pl.BlockSpec((2, _BR, 128), lambda i: (0, i, 0)),
        out_shape=jax.ShapeDtypeStruct((2, _N, 128), jnp.float32),
    )(y, st, g, b)


def _norm3_body(y_ref, st_ref, g_ref, b_ref, bt_ref,
                node_ref, pooled_ref, cnt_ref):
    mean = st_ref[0:1, :] / _N
    var = st_ref[1:2, :] / _N - mean * mean
    inv = lax.rsqrt(var + 1e-5)
    h = jnp.maximum((y_ref[...] - mean) * inv * g_ref[...] + b_ref[...], 0.0)
    node_ref[...] = h
    bt = bt_ref[...]
    io = lax.broadcasted_iota(jnp.int32, (1, _B), 1).astype(jnp.float32)
    oh = (bt == io).astype(jnp.float32)
    dn0 = (((0,), (0,)), ((), ()))
    pc = lax.dot_general(oh, h, dn0, preferred_element_type=jnp.float32)
    cc = lax.dot_general(oh, jnp.ones((_BR, _B), jnp.float32), dn0,
                         preferred_element_type=jnp.float32)

    @pl.when(pl.program_id(0) == 0)
    def _():
        pooled_ref[...] = pc
        cnt_ref[...] = cc

    @pl.when(pl.program_id(0) != 0)
    def _():
        pooled_ref[...] = pooled_ref[...] + pc
        cnt_ref[...] = cnt_ref[...] + cc


def _norm3_layer(y, st, g, b, bt):
    nb = _N // _BR
    return pl.pallas_call(
        _norm3_body,
        grid=(nb,),
        in_specs=[
            pl.BlockSpec((_BR, _H), lambda i: (i, 0)),
            pl.BlockSpec((8, _H), lambda i: (0, 0)),
            pl.BlockSpec((1, _H), lambda i: (0, 0)),
            pl.BlockSpec((1, _H), lambda i: (0, 0)),
            pl.BlockSpec((_BR, _B), lambda i: (i, 0)),
        ],
        out_specs=[
            pl.BlockSpec((_BR, _H), lambda i: (i, 0)),
            pl.BlockSpec((_B, _H), lambda i: (0, 0)),
            pl.BlockSpec((_B, _B), lambda i: (0, 0)),
        ],
        out_shape=[
            jax.ShapeDtypeStruct((_N, _H), jnp.float32),
            jax.ShapeDtypeStruct((_B, _H), jnp.float32),
            jax.ShapeDtypeStruct((_B, _B), jnp.float32),
        ],
    )(y, st, g, b, bt)


def _mlp_body(pooled_ref, cnt_ref, rad_ref, rg_ref, rb_ref,
              wc1_ref, bc1_ref, wc2_ref, bc2_ref, wc3_ref, bc3_ref,
              we_ref, be_ref, logits_ref, emb_ref):
    dn = (((1,), (1,)), ((), ()))
    ge = pooled_ref[...] / jnp.maximum(cnt_ref[:, 0:1], 1.0)
    rad = rad_ref[...]
    m = jnp.mean(rad, axis=0, keepdims=True)
    v = jnp.mean(rad * rad, axis=0, keepdims=True) - m * m
    rn = (rad - m) * lax.rsqrt(v + 1e-5) * rg_ref[...] + rb_ref[...]
    fused = jnp.concatenate([ge, rn], axis=1)
    h1 = jnp.maximum(
        lax.dot_general(fused, wc1_ref[...], dn,
                        preferred_element_type=jnp.float32) + bc1_ref[...], 0.0)
    h2 = jnp.maximum(
        lax.dot_general(h1, wc2_ref[...], dn,
                        preferred_element_type=jnp.float32) + bc2_ref[...], 0.0)
    logits_ref[...] = (
        lax.dot_general(h2, wc3_ref[...], dn,
                        preferred_element_type=jnp.float32) + bc3_ref[...])
    emb_ref[...] = (
        lax.dot_general(fused, we_ref[...], dn,
                        preferred_element_type=jnp.float32) + be_ref[...])


def _mlp_head(pooled, cnt, rad, rg, rb, wc1, bc1, wc2, bc2, wc3, bc3, we, be):
    return pl.pallas_call(
        _mlp_body,
        out_shape=[
            jax.ShapeDtypeStruct((_B, 2), jnp.float32),
            jax.ShapeDtypeStruct((_B, _H + 64), jnp.float32),
        ],
    )(pooled, cnt, rad, rg, rb, wc1, bc1, wc2, bc2, wc3, bc3, we, be)


def kernel(x, edge_index, batch, radiomics,
           Wl0, bl0, Wr0, g0, b0, Wl1, bl1, Wr1, g1, b1,
           Wl2, bl2, Wr2, g2, b2, rg, rb,
           Wc1, bc1, Wc2, bc2, Wc3, bc3, We, be):
    f32 = jnp.float32
    src = edge_index[0]
    dst = edge_index[1]
    bt = jnp.broadcast_to(batch[:, None].astype(f32), (_N, _B))

    npad_sl = _RS * _K - _E
    npad_es = _RE * _K - _E // 2
    pad_src_sl = jnp.arange(npad_sl, dtype=jnp.int32) % _N
    pad_dst_sl = _N + jnp.arange(npad_sl, dtype=jnp.int32) % (_NP - _N)
    pad_src_es = pad_src_sl[:npad_es]
    pad_dst_es = pad_dst_sl[:npad_es]

    src_p = jnp.concatenate([src, pad_src_sl])
    srcs_sl = jnp.concatenate([src_p, src_p + _N])
    dst_sl = jnp.concatenate([dst, pad_dst_sl])

    eh = _E // 2
    src_es = jnp.concatenate([src[:eh], pad_src_es, src[eh:], pad_src_es])
    dst_es = jnp.concatenate([dst[:eh], pad_dst_es, dst[eh:], pad_dst_es])

    agg0r, degr = _make_sc_esplit()(x, src_es, dst_es)
    agg0 = agg0r.reshape(2, _NP, 128)
    degp = degr.reshape(2, _NP, 128)

    y0, st0 = _mm_layer(agg0, x, degp, Wl0, bl0.reshape(1, _H), Wr0, "sum")
    h1 = _norm_layer(y0, st0, g0.reshape(1, _H), b0.reshape(1, _H))

    agg1 = _make_sc_slab()(h1.reshape(2 * _N, 128), srcs_sl, dst_sl)
    y1, st1 = _mm_layer(agg1.reshape(2, _NP, 128), h1,
                        degp, Wl1, bl1.reshape(1, _H), Wr1, "concat")
    h2 = _norm_layer(y1, st1, g1.reshape(1, _H), b1.reshape(1, _H))

    agg2 = _make_sc_slab()(h2.reshape(2 * _N, 128), srcs_sl, dst_sl)
    y2, st2 = _mm_layer(agg2.reshape(2, _NP, 128), h2,
                        degp, Wl2, bl2.reshape(1, _H), Wr2, "concat")
    node_emb, pooled, cnt = _norm3_layer(y2, st2, g2.reshape(1, _H),
                                         b2.reshape(1, _H), bt)

    logits, embedding = _mlp_head(
        pooled, cnt, radiomics, rg.reshape(1, 64), rb.reshape(1, 64),
        Wc1, bc1.reshape(1, 64), Wc2, bc2.reshape(1, 32),
        Wc3, bc3.reshape(1, 2), We, be.reshape(1, _H + 64))
    return (logits, embedding, node_emb)


# speedup vs baseline: 1.0195x; 1.0195x over previous
import functools

import jax
import jax.numpy as jnp
from jax import lax
from jax.experimental import pallas as pl
from jax.experimental.pallas import tpu as pltpu
import jax.experimental.pallas.tpu_sc as plsc

_N = 10000
_E = 320000
_B = 16
_H = 256
_NC = 2
_NS = 16
_K = 128
_NP = 10240
_RPT = _NP // _NS
_RS = 2560
_TRS = _RS // _NS
_RE = 1280
_TRE = _RE // _NS
_BR = 1000


def _zero_rows(buf, n):
    zero16 = jnp.zeros((16,), jnp.float32)

    def _z(i, _):
        for j in range(128 // 16):
            buf[i, pl.ds(j * 16, 16)] = zero16
        return _
    lax.fori_loop(0, n, _z, None)


def _zero_acc(acc, rows, s):
    r0 = s * _RPT
    for j in range(5):
        pltpu.sync_copy(rows.at[pl.ds(0, _K)],
                        acc.at[pl.ds(r0 + j * _K, _K)])


def _acc_writeout(acc, out, c, s):
    r0 = s * _RPT
    pltpu.sync_copy(acc.at[pl.ds(r0, _RPT)],
                    out.at[pl.ds(c * _NP + r0, _RPT)])


def _agg_pipeline(table, acc, srcs, dst, sbase, dbase, nchunks,
                  iss, ids, isems, rows, gsems, dhook=None, coff=None):
    ssems = gsems[2:]
    gsems = gsems[:2]

    def ld(chunk, j):
        off = chunk * _K
        pltpu.async_copy(srcs.at[pl.ds(sbase + off, _K)], iss[j], isems[j])
        pltpu.async_copy(dst.at[pl.ds(dbase + off, _K)], ids[j], isems[j])

    def wld(j):
        pltpu.make_async_copy(srcs.at[pl.ds(0, _K)], iss[j], isems[j]).wait()
        pltpu.make_async_copy(dst.at[pl.ds(0, _K)], ids[j], isems[j]).wait()

    def g(j, slot):
        if coff is not None:
            for k in range(8):
                iss[j][pl.ds(k * 16, 16)] = iss[j][pl.ds(k * 16, 16)] + coff
        pltpu.async_copy(table.at[iss[j]], rows[slot], gsems[slot])

    def wg(slot):
        pltpu.make_async_copy(table.at[pl.ds(0, _K)], rows[slot],
                              gsems[slot]).wait()

    def sc(j, slot):
        pltpu.async_copy(rows[slot], acc.at[ids[j]], ssems[slot], add=True)
        if dhook is not None:
            dhook(j)

    def wsc(j, slot):
        pltpu.make_async_copy(rows[slot], acc.at[ids[j]], ssems[slot]).wait()

    ld(0, 0)
    ld(1, 1)
    ld(2, 2)
    wld(0)
    g(0, 0)

    nb = nchunks // 4

    def _body(t, carry):
        b = 4 * t
        nxt = t < nb - 1
        ld(b + 3, 3)
        wld(1)

        @pl.when(t > 0)
        def _():
            wsc(3, 1)
        g(1, 1)
        wg(0)
        sc(0, 0)

        @pl.when(nxt)
        def _():
            ld(b + 4, 0)
        wld(2)
        wsc(0, 0)
        g(2, 0)
        wg(1)
        sc(1, 1)

        @pl.when(nxt)
        def _():
            ld(b + 5, 1)
        wld(3)
        wsc(1, 1)
        g(3, 1)
        wg(0)
        sc(2, 0)

        @pl.when(nxt)
        def _():
            ld(b + 6, 2)
            wld(0)
            wsc(2, 0)
            g(0, 0)
        wg(1)
        sc(3, 1)
        return carry
    lax.fori_loop(0, nb, _body, None)
    wsc(2, 0)
    wsc(3, 1)


def _sc_agg_common(table, srcs, dst, agg, sbase, dbase, nchunks, refs):
    (acc, i0, i1, i2, i3, d0, d1, d2, d3, r0, r1,
     m0, m1, m2, m3, gs0, gs1, ss0, ss1) = refs
    c = lax.axis_index("c")
    s = lax.axis_index("s")
    _zero_rows(r0, _K)
    _zero_acc(acc, r0, s)
    plsc.subcore_barrier()
    _agg_pipeline(table, acc, srcs, dst, sbase, dbase, nchunks,
                  (i0, i1, i2, i3), (d0, d1, d2, d3), (m0, m1, m2, m3),
                  (r0, r1), (gs0, gs1, ss0, ss1), coff=c * _N)
    plsc.subcore_barrier()
    _acc_writeout(acc, agg, c, s)


def _sc_slab_body(table, srcs, dst, agg, *refs):
    s = lax.axis_index("s")
    base = (s * _TRS) * _K
    _sc_agg_common(table, srcs, dst, agg, base, base, _TRS, refs)


def _sc_esplit_body(table, srcs, dst, agg, deg, *refs):
    (acc, dacc, i0, i1, i2, i3, d0, d1, d2, d3, r0, r1,
     m0, m1, m2, m3, gs0, gs1, ss0, ss1, ones, dbuf) = refs
    c = lax.axis_index("c")
    s = lax.axis_index("s")
    zero16 = jnp.zeros((16,), jnp.float32)
    _zero_rows(r0, _K)
    _zero_acc(acc, r0, s)
    r0b = s * _RPT
    for j in range(8):
        ones[pl.ds(j * 16, 16)] = zero16
    for j in range(5):
        pltpu.sync_copy(ones, dacc.at[pl.ds(r0b + j * _K, _K)])
    one16 = jnp.ones((16,), jnp.float32)
    for j in range(8):
        ones[pl.ds(j * 16, 16)] = one16
    plsc.subcore_barrier()

    base = (c * _RE + s * _TRE) * _K
    ids = (d0, d1, d2, d3)

    def dhook(j):
        pltpu.sync_copy(ones, dacc.at[ids[j]], add=True)

    _agg_pipeline(table, acc, srcs, dst, base, base, _TRE,
                  (i0, i1, i2, i3), ids, (m0, m1, m2, m3),
                  (r0, r1), (gs0, gs1, ss0, ss1), dhook)
    plsc.subcore_barrier()
    _acc_writeout(acc, agg, c, s)

    pltpu.sync_copy(dacc.at[pl.ds(r0b, _RPT)], dbuf)
    for blk in range(5):
        def _brow(g, carry):
            v16 = dbuf[pl.ds(blk * _K + g * 16, 16)]
            for l in range(16):
                row = jnp.full((16,), v16[l], jnp.float32)
                for j in range(8):
                    r0[g * 16 + l, pl.ds(j * 16, 16)] = row
            return carry
        lax.fori_loop(0, 8, _brow, None)
        pltpu.sync_copy(r0, deg.at[pl.ds(c * _NP + r0b + blk * _K, _K)])


def _sc_mesh():
    return plsc.VectorSubcoreMesh(core_axis_name="c", subcore_axis_name="s",
                                  num_cores=_NC, num_subcores=_NS)


def _agg_scratch():
    return (
        (pltpu.VMEM_SHARED((_NP, 128), jnp.float32),)
        + (pltpu.VMEM((_K,), jnp.int32),) * 8
        + (pltpu.VMEM((_K, 128), jnp.float32),) * 2
        + (pltpu.SemaphoreType.DMA,) * 8
    )


_SC_OUT = jax.ShapeDtypeStruct((_NC * _NP, 128), jnp.float32)


@functools.lru_cache(maxsize=None)
def _make_sc_slab():
    return pl.kernel(_sc_slab_body, out_type=_SC_OUT,
                     mesh=_sc_mesh(), scratch_types=_agg_scratch())


@functools.lru_cache(maxsize=None)
def _make_sc_esplit():
    scratch = (
        (pltpu.VMEM_SHARED((_NP, 128), jnp.float32),
         pltpu.VMEM_SHARED((_NP,), jnp.float32))
        + (pltpu.VMEM((_K,), jnp.int32),) * 8
        + (pltpu.VMEM((_K, 128), jnp.float32),) * 2
        + (pltpu.SemaphoreType.DMA,) * 8
        + (pltpu.VMEM((_K,), jnp.float32),
           pltpu.VMEM((_RPT,), jnp.float32))
    )
    return pl.kernel(_sc_esplit_body, out_type=(_SC_OUT, _SC_OUT),
                     mesh=_sc_mesh(), scratch_types=scratch)


def _mm_body(mode, agg_ref, h_ref, deg_ref, wl_ref, bl_ref, wr_ref,
             y_ref, st_ref):
    dn = (((1,), (1,)), ((), ()))
    if mode == "sum":
        aggc = agg_ref[0] + agg_ref[1]
        hc = h_ref[...]
    else:
        aggc = jnp.concatenate([agg_ref[0], agg_ref[1]], axis=1)
        hc = jnp.concatenate([h_ref[0], h_ref[1]], axis=1)
    yl = lax.dot_general(aggc, wl_ref[...], dn,
                         preferred_element_type=jnp.float32)
    d = deg_ref[0, :, 0:1] + deg_ref[1, :, 0:1]
    rdeg = 1.0 / jnp.maximum(d, 1.0)
    y = (yl * rdeg + bl_ref[...]
         + lax.dot_general(hc, wr_ref[...], dn,
                           preferred_element_type=jnp.float32))
    y_ref[...] = y
    s1 = jnp.sum(y, axis=0, keepdims=True)
    s2 = jnp.sum(y * y, axis=0, keepdims=True)
    blk = jnp.concatenate([s1, s2, jnp.zeros((6, _H), jnp.float32)], axis=0)

    @pl.when(pl.program_id(0) == 0)
    def _():
        st_ref[...] = blk

    @pl.when(pl.program_id(0) != 0)
    def _():
        st_ref[...] = st_ref[...] + blk


def _mm_layer(agg, h, deg, wl, bl, wr, mode):
    nb = _N // _BR
    din = wl.shape[1]
    if mode == "sum":
        h_spec = pl.BlockSpec((_BR, 128), lambda i: (i, 0))
    else:
        h_spec = pl.BlockSpec((2, _BR, 128), lambda i: (0, i, 0))
    return pl.pallas_call(
        functools.partial(_mm_body, mode),
        grid=(nb,),
        in_specs=[
            pl.BlockSpec((2, _BR, 128), lambda i: (0, i, 0)),
            h_spec,
            pl.BlockSpec((2, _BR, 128), lambda i: (0, i, 0)),
            pl.BlockSpec((_H, din), lambda i: (0, 0)),
            pl.BlockSpec((1, _H), lambda i: (0, 0)),
            pl.BlockSpec((_H, din), lambda i: (0, 0)),
        ],
        out_specs=[
            pl.BlockSpec((_BR, _H), lambda i: (i, 0)),
            pl.BlockSpec((8, _H), lambda i: (0, 0)),
        ],
        out_shape=[
            jax.ShapeDtypeStruct((_N, _H), jnp.float32),
            jax.ShapeDtypeStruct((8, _H), jnp.float32),
        ],
    )(agg, h, deg, wl, bl, wr)


def _norm_body(y_ref, st_ref, g_ref, b_ref, out_ref):
    mean = st_ref[0:1, :] / _N
    var = st_ref[1:2, :] / _N - mean * mean
    inv = lax.rsqrt(var + 1e-5)
    h = jnp.maximum((y_ref[...] - mean) * inv * g_ref[...] + b_ref[...], 0.0)
    out_ref[0] = h[:, 0:128]
    out_ref[1] = h[:, 128:256]


def _norm_layer(y, st, g, b):
    nb = _N // _BR
    return pl.pallas_call(
        _norm_body,
        grid=(nb,),
        in_specs=[
            pl.BlockSpec((_BR, _H), lambda i: (i, 0)),
            pl.BlockSpec((8, _H), lambda i: (0, 0)),
            pl.BlockSpec((1, _H), lambda i: (0, 0)),
            pl.BlockSpec((1, _H), lambda i: (0, 0)),
        ],
        out_specs=pl.BlockSpec((2, _BR, 128), lambda i: (0, i, 0)),
        out_shape=jax.ShapeDtypeStruct((2, _N, 128), jnp.float32),
    )(y, st, g, b)


def _norm3_body(y_ref, st_ref, g_ref, b_ref, bt_ref,
                node_ref, pooled_ref, cnt_ref):
    mean = st_ref[0:1, :] / _N
    var = st_ref[1:2, :] / _N - mean * mean
    inv = lax.rsqrt(var + 1e-5)
    h = jnp.maximum((y_ref[...] - mean) * inv * g_ref[...] + b_ref[...], 0.0)
    node_ref[...] = h
    bt = bt_ref[...]
    io = lax.broadcasted_iota(jnp.int32, (1, _B), 1)
    oh = (bt == io).astype(jnp.float32)
    dn0 = (((0,), (0,)), ((), ()))
    pc = lax.dot_general(oh, h, dn0, preferred_element_type=jnp.float32)
    cc = lax.dot_general(oh, jnp.ones((_BR, _B), jnp.float32), dn0,
                         preferred_element_type=jnp.float32)

    @pl.when(pl.program_id(0) == 0)
    def _():
        pooled_ref[...] = pc
        cnt_ref[...] = cc

    @pl.when(pl.program_id(0) != 0)
    def _():
        pooled_ref[...] = pooled_ref[...] + pc
        cnt_ref[...] = cnt_ref[...] + cc


def _norm3_layer(y, st, g, b, bt):
    nb = _N // _BR
    return pl.pallas_call(
        _norm3_body,
        grid=(nb,),
        in_specs=[
            pl.BlockSpec((_BR, _H), lambda i: (i, 0)),
            pl.BlockSpec((8, _H), lambda i: (0, 0)),
            pl.BlockSpec((1, _H), lambda i: (0, 0)),
            pl.BlockSpec((1, _H), lambda i: (0, 0)),
            pl.BlockSpec((_BR, 1), lambda i: (i, 0)),
        ],
        out_specs=[
            pl.BlockSpec((_BR, _H), lambda i: (i, 0)),
            pl.BlockSpec((_B, _H), lambda i: (0, 0)),
            pl.BlockSpec((_B, _B), lambda i: (0, 0)),
        ],
        out_shape=[
            jax.ShapeDtypeStruct((_N, _H), jnp.float32),
            jax.ShapeDtypeStruct((_B, _H), jnp.float32),
            jax.ShapeDtypeStruct((_B, _B), jnp.float32),
        ],
    )(y, st, g, b, bt)


def _mlp_body(pooled_ref, cnt_ref, rad_ref, rg_ref, rb_ref,
              wc1_ref, bc1_ref, wc2_ref, bc2_ref, wc3_ref, bc3_ref,
              we_ref, be_ref, logits_ref, emb_ref):
    dn = (((1,), (1,)), ((), ()))
    ge = pooled_ref[...] / jnp.maximum(cnt_ref[:, 0:1], 1.0)
    rad = rad_ref[...]
    m = jnp.mean(rad, axis=0, keepdims=True)
    v = jnp.mean(rad * rad, axis=0, keepdims=True) - m * m
    rn = (rad - m) * lax.rsqrt(v + 1e-5) * rg_ref[...] + rb_ref[...]
    fused = jnp.concatenate([ge, rn], axis=1)
    h1 = jnp.maximum(
        lax.dot_general(fused, wc1_ref[...], dn,
                        preferred_element_type=jnp.float32) + bc1_ref[...], 0.0)
    h2 = jnp.maximum(
        lax.dot_general(h1, wc2_ref[...], dn,
                        preferred_element_type=jnp.float32) + bc2_ref[...], 0.0)
    logits_ref[...] = (
        lax.dot_general(h2, wc3_ref[...], dn,
                        preferred_element_type=jnp.float32) + bc3_ref[...])
    emb_ref[...] = (
        lax.dot_general(fused, we_ref[...], dn,
                        preferred_element_type=jnp.float32) + be_ref[...])


def _mlp_head(pooled, cnt, rad, rg, rb, wc1, bc1, wc2, bc2, wc3, bc3, we, be):
    return pl.pallas_call(
        _mlp_body,
        out_shape=[
            jax.ShapeDtypeStruct((_B, 2), jnp.float32),
            jax.ShapeDtypeStruct((_B, _H + 64), jnp.float32),
        ],
    )(pooled, cnt, rad, rg, rb, wc1, bc1, wc2, bc2, wc3, bc3, we, be)


def kernel(x, edge_index, batch, radiomics,
           Wl0, bl0, Wr0, g0, b0, Wl1, bl1, Wr1, g1, b1,
           Wl2, bl2, Wr2, g2, b2, rg, rb,
           Wc1, bc1, Wc2, bc2, Wc3, bc3, We, be):
    src = edge_index[0]
    dst = edge_index[1]
    bt = batch.reshape(_N, 1)

    npad = _RS * _K - _E
    src_p = jnp.concatenate([src, jnp.arange(npad, dtype=jnp.int32) % _N])
    dst_p = jnp.concatenate(
        [dst, _N + jnp.arange(npad, dtype=jnp.int32) % (_NP - _N)])

    agg0r, degr = _make_sc_esplit()(x, src_p, dst_p)
    agg0 = agg0r.reshape(2, _NP, 128)
    degp = degr.reshape(2, _NP, 128)

    y0, st0 = _mm_layer(agg0, x, degp, Wl0, bl0.reshape(1, _H), Wr0, "sum")
    h1 = _norm_layer(y0, st0, g0.reshape(1, _H), b0.reshape(1, _H))

    agg1 = _make_sc_slab()(h1.reshape(2 * _N, 128), src_p, dst_p)
    y1, st1 = _mm_layer(agg1.reshape(2, _NP, 128), h1,
                        degp, Wl1, bl1.reshape(1, _H), Wr1, "concat")
    h2 = _norm_layer(y1, st1, g1.reshape(1, _H), b1.reshape(1, _H))

    agg2 = _make_sc_slab()(h2.reshape(2 * _N, 128), src_p, dst_p)
    y2, st2 = _mm_layer(agg2.reshape(2, _NP, 128), h2,
                        degp, Wl2, bl2.reshape(1, _H), Wr2, "concat")
    node_emb, pooled, cnt = _norm3_layer(y2, st2, g2.reshape(1, _H),
                                         b2.reshape(1, _H), bt)

    logits, embedding = _mlp_head(
        pooled, cnt, radiomics, rg.reshape(1, 64), rb.reshape(1, 64),
        Wc1, bc1.reshape(1, 64), Wc2, bc2.reshape(1, 32),
        Wc3, bc3.reshape(1, 2), We, be.reshape(1, _H + 64))
    return (logits, embedding, node_emb)


# speedup vs baseline: 1.0386x; 1.0187x over previous
import functools

import jax
import jax.numpy as jnp
from jax import lax
from jax.experimental import pallas as pl
from jax.experimental.pallas import tpu as pltpu
import jax.experimental.pallas.tpu_sc as plsc

_N = 10000
_E = 320000
_B = 16
_H = 256
_NC = 2
_NS = 16
_K = 128
_NP = 10240
_RPT = _NP // _NS
_RS = 2560
_TRS = _RS // _NS
_RE = 1280
_TRE = _RE // _NS
_BR = 2000


def _zero_rows(buf, n):
    zero16 = jnp.zeros((16,), jnp.float32)

    def _z(i, _):
        for j in range(128 // 16):
            buf[i, pl.ds(j * 16, 16)] = zero16
        return _
    lax.fori_loop(0, n, _z, None)


def _zero_acc(acc, rows, s):
    r0 = s * _RPT
    for j in range(5):
        pltpu.sync_copy(rows.at[pl.ds(0, _K)],
                        acc.at[pl.ds(r0 + j * _K, _K)])


def _acc_writeout(acc, out, c, s):
    r0 = s * _RPT
    pltpu.sync_copy(acc.at[pl.ds(r0, _RPT)],
                    out.at[pl.ds(c * _NP + r0, _RPT)])


def _agg_pipeline(table, acc, srcs, dst, sbase, dbase, nchunks,
                  iss, ids, isems, rows, gsems, dhook=None, coff=None):
    ssems = gsems[2:]
    gsems = gsems[:2]

    def ld(chunk, j):
        off = chunk * _K
        pltpu.async_copy(srcs.at[pl.ds(sbase + off, _K)], iss[j], isems[j])
        pltpu.async_copy(dst.at[pl.ds(dbase + off, _K)], ids[j], isems[j])

    def wld(j):
        pltpu.make_async_copy(srcs.at[pl.ds(0, _K)], iss[j], isems[j]).wait()
        pltpu.make_async_copy(dst.at[pl.ds(0, _K)], ids[j], isems[j]).wait()

    def g(j, slot):
        if coff is not None:
            for k in range(8):
                iss[j][pl.ds(k * 16, 16)] = iss[j][pl.ds(k * 16, 16)] + coff
        pltpu.async_copy(table.at[iss[j]], rows[slot], gsems[slot])

    def wg(slot):
        pltpu.make_async_copy(table.at[pl.ds(0, _K)], rows[slot],
                              gsems[slot]).wait()

    def sc(j, slot):
        pltpu.async_copy(rows[slot], acc.at[ids[j]], ssems[slot], add=True)
        if dhook is not None:
            dhook(j)

    def wsc(j, slot):
        pltpu.make_async_copy(rows[slot], acc.at[ids[j]], ssems[slot]).wait()

    ld(0, 0)
    ld(1, 1)
    ld(2, 2)
    wld(0)
    g(0, 0)

    nb = nchunks // 4

    def _body(t, carry):
        b = 4 * t
        nxt = t < nb - 1
        ld(b + 3, 3)
        wld(1)

        @pl.when(t > 0)
        def _():
            wsc(3, 1)
        g(1, 1)
        wg(0)
        sc(0, 0)

        @pl.when(nxt)
        def _():
            ld(b + 4, 0)
        wld(2)
        wsc(0, 0)
        g(2, 0)
        wg(1)
        sc(1, 1)

        @pl.when(nxt)
        def _():
            ld(b + 5, 1)
        wld(3)
        wsc(1, 1)
        g(3, 1)
        wg(0)
        sc(2, 0)

        @pl.when(nxt)
        def _():
            ld(b + 6, 2)
            wld(0)
            wsc(2, 0)
            g(0, 0)
        wg(1)
        sc(3, 1)
        return carry
    lax.fori_loop(0, nb, _body, None)
    wsc(2, 0)
    wsc(3, 1)


def _sc_agg_common(table, srcs, dst, agg, sbase, dbase, nchunks, refs):
    (acc, i0, i1, i2, i3, d0, d1, d2, d3, r0, r1,
     m0, m1, m2, m3, gs0, gs1, ss0, ss1) = refs
    c = lax.axis_index("c")
    s = lax.axis_index("s")
    _zero_rows(r0, _K)
    _zero_acc(acc, r0, s)
    plsc.subcore_barrier()
    _agg_pipeline(table, acc, srcs, dst, sbase, dbase, nchunks,
                  (i0, i1, i2, i3), (d0, d1, d2, d3), (m0, m1, m2, m3),
                  (r0, r1), (gs0, gs1, ss0, ss1), coff=c * _N)
    plsc.subcore_barrier()
    _acc_writeout(acc, agg, c, s)


def _sc_slab_body(table, srcs, dst, agg, *refs):
    s = lax.axis_index("s")
    base = (s * _TRS) * _K
    _sc_agg_common(table, srcs, dst, agg, base, base, _TRS, refs)


def _sc_esplit_body(table, srcs, dst, agg, deg, *refs):
    (acc, dacc, i0, i1, i2, i3, d0, d1, d2, d3, r0, r1,
     m0, m1, m2, m3, gs0, gs1, ss0, ss1, ones, dbuf) = refs
    c = lax.axis_index("c")
    s = lax.axis_index("s")
    zero16 = jnp.zeros((16,), jnp.float32)
    _zero_rows(r0, _K)
    _zero_acc(acc, r0, s)
    r0b = s * _RPT
    for j in range(8):
        ones[pl.ds(j * 16, 16)] = zero16
    for j in range(5):
        pltpu.sync_copy(ones, dacc.at[pl.ds(r0b + j * _K, _K)])
    one16 = jnp.ones((16,), jnp.float32)
    for j in range(8):
        ones[pl.ds(j * 16, 16)] = one16
    plsc.subcore_barrier()

    base = (c * _RE + s * _TRE) * _K
    ids = (d0, d1, d2, d3)

    def dhook(j):
        pltpu.sync_copy(ones, dacc.at[ids[j]], add=True)

    _agg_pipeline(table, acc, srcs, dst, base, base, _TRE,
                  (i0, i1, i2, i3), ids, (m0, m1, m2, m3),
                  (r0, r1), (gs0, gs1, ss0, ss1), dhook)
    plsc.subcore_barrier()
    _acc_writeout(acc, agg, c, s)

    pltpu.sync_copy(dacc.at[pl.ds(r0b, _RPT)], dbuf)
    for blk in range(5):
        def _brow(g, carry):
            v16 = dbuf[pl.ds(blk * _K + g * 16, 16)]
            for l in range(16):
                row = jnp.full((16,), v16[l], jnp.float32)
                for j in range(8):
                    r0[g * 16 + l, pl.ds(j * 16, 16)] = row
            return carry
        lax.fori_loop(0, 8, _brow, None)
        pltpu.sync_copy(r0, deg.at[pl.ds(c * _NP + r0b + blk * _K, _K)])


def _sc_mesh():
    return plsc.VectorSubcoreMesh(core_axis_name="c", subcore_axis_name="s",
                                  num_cores=_NC, num_subcores=_NS)


def _agg_scratch():
    return (
        (pltpu.VMEM_SHARED((_NP, 128), jnp.float32),)
        + (pltpu.VMEM((_K,), jnp.int32),) * 8
        + (pltpu.VMEM((_K, 128), jnp.float32),) * 2
        + (pltpu.SemaphoreType.DMA,) * 8
    )


_SC_OUT = jax.ShapeDtypeStruct((_NC * _NP, 128), jnp.float32)


@functools.lru_cache(maxsize=None)
def _make_sc_slab():
    return pl.kernel(_sc_slab_body, out_type=_SC_OUT,
                     mesh=_sc_mesh(), scratch_types=_agg_scratch())


@functools.lru_cache(maxsize=None)
def _make_sc_esplit():
    scratch = (
        (pltpu.VMEM_SHARED((_NP, 128), jnp.float32),
         pltpu.VMEM_SHARED((_NP,), jnp.float32))
        + (pltpu.VMEM((_K,), jnp.int32),) * 8
        + (pltpu.VMEM((_K, 128), jnp.float32),) * 2
        + (pltpu.SemaphoreType.DMA,) * 8
        + (pltpu.VMEM((_K,), jnp.float32),
           pltpu.VMEM((_RPT,), jnp.float32))
    )
    return pl.kernel(_sc_esplit_body, out_type=(_SC_OUT, _SC_OUT),
                     mesh=_sc_mesh(), scratch_types=scratch)


def _mm_body(mode, agg_ref, h_ref, deg_ref, wl_ref, bl_ref, wr_ref,
             y_ref, st_ref):
    dn = (((1,), (1,)), ((), ()))
    if mode == "sum":
        aggc = agg_ref[0] + agg_ref[1]
        hc = h_ref[...]
    else:
        aggc = jnp.concatenate([agg_ref[0], agg_ref[1]], axis=1)
        hc = jnp.concatenate([h_ref[0], h_ref[1]], axis=1)
    yl = lax.dot_general(aggc, wl_ref[...], dn,
                         preferred_element_type=jnp.float32)
    d = deg_ref[0, :, 0:1] + deg_ref[1, :, 0:1]
    rdeg = 1.0 / jnp.maximum(d, 1.0)
    y = (yl * rdeg + bl_ref[...]
         + lax.dot_general(hc, wr_ref[...], dn,
                           preferred_element_type=jnp.float32))
    y_ref[...] = y
    s1 = jnp.sum(y, axis=0, keepdims=True)
    s2 = jnp.sum(y * y, axis=0, keepdims=True)
    blk = jnp.concatenate([s1, s2, jnp.zeros((6, _H), jnp.float32)], axis=0)

    @pl.when(pl.program_id(0) == 0)
    def _():
        st_ref[...] = blk

    @pl.when(pl.program_id(0) != 0)
    def _():
        st_ref[...] = st_ref[...] + blk


def _mm_layer(agg, h, deg, wl, bl, wr, mode):
    nb = _N // _BR
    din = wl.shape[1]
    if mode == "sum":
        h_spec = pl.BlockSpec((_BR, 128), lambda i: (i, 0))
    else:
        h_spec = pl.BlockSpec((2, _BR, 128), lambda i: (0, i, 0))
    return pl.pallas_call(
        functools.partial(_mm_body, mode),
        grid=(nb,),
        in_specs=[
            pl.BlockSpec((2, _BR, 128), lambda i: (0, i, 0)),
            h_spec,
            pl.BlockSpec((2, _BR, 128), lambda i: (0, i, 0)),
            pl.BlockSpec((_H, din), lambda i: (0, 0)),
            pl.BlockSpec((1, _H), lambda i: (0, 0)),
            pl.BlockSpec((_H, din), lambda i: (0, 0)),
        ],
        out_specs=[
            pl.BlockSpec((_BR, _H), lambda i: (i, 0)),
            pl.BlockSpec((8, _H), lambda i: (0, 0)),
        ],
        out_shape=[
            jax.ShapeDtypeStruct((_N, _H), jnp.float32),
            jax.ShapeDtypeStruct((8, _H), jnp.float32),
        ],
    )(agg, h, deg, wl, bl, wr)


def _norm_body(y_ref, st_ref, g_ref, b_ref, out_ref):
    mean = st_ref[0:1, :] / _N
    var = st_ref[1:2, :] / _N - mean * mean
    inv = lax.rsqrt(var + 1e-5)
    h = jnp.maximum((y_ref[...] - mean) * inv * g_ref[...] + b_ref[...], 0.0)
    out_ref[0] = h[:, 0:128]
    out_ref[1] = h[:, 128:256]


def _norm_layer(y, st, g, b):
    nb = _N // _BR
    return pl.pallas_call(
        _norm_body,
        grid=(nb,),
        in_specs=[
            pl.BlockSpec((_BR, _H), lambda i: (i, 0)),
            pl.BlockSpec((8, _H), lambda i: (0, 0)),
            pl.BlockSpec((1, _H), lambda i: (0, 0)),
            pl.BlockSpec((1, _H), lambda i: (0, 0)),
        ],
        out_specs=pl.BlockSpec((2, _BR, 128), lambda i: (0, i, 0)),
        out_shape=jax.ShapeDtypeStruct((2, _N, 128), jnp.float32),
    )(y, st, g, b)


def _norm3_body(y_ref, st_ref, g_ref, b_ref, bt_ref,
                node_ref, pooled_ref, cnt_ref):
    mean = st_ref[0:1, :] / _N
    var = st_ref[1:2, :] / _N - mean * mean
    inv = lax.rsqrt(var + 1e-5)
    h = jnp.maximum((y_ref[...] - mean) * inv * g_ref[...] + b_ref[...], 0.0)
    node_ref[...] = h
    bt = bt_ref[...]
    io = lax.broadcasted_iota(jnp.int32, (1, _B), 1)
    oh = (bt == io).astype(jnp.float32)
    dn0 = (((0,), (0,)), ((), ()))
    pc = lax.dot_general(oh, h, dn0, preferred_element_type=jnp.float32)
    cc = lax.dot_general(oh, jnp.ones((_BR, _B), jnp.float32), dn0,
                         preferred_element_type=jnp.float32)

    @pl.when(pl.program_id(0) == 0)
    def _():
        pooled_ref[...] = pc
        cnt_ref[...] = cc

    @pl.when(pl.program_id(0) != 0)
    def _():
        pooled_ref[...] = pooled_ref[...] + pc
        cnt_ref[...] = cnt_ref[...] + cc


def _norm3_layer(y, st, g, b, bt):
    nb = _N // _BR
    return pl.pallas_call(
        _norm3_body,
        grid=(nb,),
        in_specs=[
            pl.BlockSpec((_BR, _H), lambda i: (i, 0)),
            pl.BlockSpec((8, _H), lambda i: (0, 0)),
            pl.BlockSpec((1, _H), lambda i: (0, 0)),
            pl.BlockSpec((1, _H), lambda i: (0, 0)),
            pl.BlockSpec((_BR, 1), lambda i: (i, 0)),
        ],
        out_specs=[
            pl.BlockSpec((_BR, _H), lambda i: (i, 0)),
            pl.BlockSpec((_B, _H), lambda i: (0, 0)),
            pl.BlockSpec((_B, _B), lambda i: (0, 0)),
        ],
        out_shape=[
            jax.ShapeDtypeStruct((_N, _H), jnp.float32),
            jax.ShapeDtypeStruct((_B, _H), jnp.float32),
            jax.ShapeDtypeStruct((_B, _B), jnp.float32),
        ],
    )(y, st, g, b, bt)


def _mlp_body(pooled_ref, cnt_ref, rad_ref, rg_ref, rb_ref,
              wc1_ref, bc1_ref, wc2_ref, bc2_ref, wc3_ref, bc3_ref,
              we_ref, be_ref, logits_ref, emb_ref):
    dn = (((1,), (1,)), ((), ()))
    ge = pooled_ref[...] / jnp.maximum(cnt_ref[:, 0:1], 1.0)
    rad = rad_ref[...]
    m = jnp.mean(rad, axis=0, keepdims=True)
    v = jnp.mean(rad * rad, axis=0, keepdims=True) - m * m
    rn = (rad - m) * lax.rsqrt(v + 1e-5) * rg_ref[...] + rb_ref[...]
    fused = jnp.concatenate([ge, rn], axis=1)
    h1 = jnp.maximum(
        lax.dot_general(fused, wc1_ref[...], dn,
                        preferred_element_type=jnp.float32) + bc1_ref[...], 0.0)
    h2 = jnp.maximum(
        lax.dot_general(h1, wc2_ref[...], dn,
                        preferred_element_type=jnp.float32) + bc2_ref[...], 0.0)
    logits_ref[...] = (
        lax.dot_general(h2, wc3_ref[...], dn,
                        preferred_element_type=jnp.float32) + bc3_ref[...])
    emb_ref[...] = (
        lax.dot_general(fused, we_ref[...], dn,
                        preferred_element_type=jnp.float32) + be_ref[...])


def _mlp_head(pooled, cnt, rad, rg, rb, wc1, bc1, wc2, bc2, wc3, bc3, we, be):
    return pl.pallas_call(
        _mlp_body,
        out_shape=[
            jax.ShapeDtypeStruct((_B, 2), jnp.float32),
            jax.ShapeDtypeStruct((_B, _H + 64), jnp.float32),
        ],
    )(pooled, cnt, rad, rg, rb, wc1, bc1, wc2, bc2, wc3, bc3, we, be)


def kernel(x, edge_index, batch, radiomics,
           Wl0, bl0, Wr0, g0, b0, Wl1, bl1, Wr1, g1, b1,
           Wl2, bl2, Wr2, g2, b2, rg, rb,
           Wc1, bc1, Wc2, bc2, Wc3, bc3, We, be):
    src = edge_index[0]
    dst = edge_index[1]
    bt = batch.reshape(_N, 1)

    npad = _RS * _K - _E
    src_p = jnp.concatenate([src, jnp.arange(npad, dtype=jnp.int32) % _N])
    dst_p = jnp.concatenate(
        [dst, _N + jnp.arange(npad, dtype=jnp.int32) % (_NP - _N)])

    agg0r, degr = _make_sc_esplit()(x, src_p, dst_p)
    agg0 = agg0r.reshape(2, _NP, 128)
    degp = degr.reshape(2, _NP, 128)

    y0, st0 = _mm_layer(agg0, x, degp, Wl0, bl0.reshape(1, _H), Wr0, "sum")
    h1 = _norm_layer(y0, st0, g0.reshape(1, _H), b0.reshape(1, _H))

    agg1 = _make_sc_slab()(h1.reshape(2 * _N, 128), src_p, dst_p)
    y1, st1 = _mm_layer(agg1.reshape(2, _NP, 128), h1,
                        degp, Wl1, bl1.reshape(1, _H), Wr1, "concat")
    h2 = _norm_layer(y1, st1, g1.reshape(1, _H), b1.reshape(1, _H))

    agg2 = _make_sc_slab()(h2.reshape(2 * _N, 128), src_p, dst_p)
    y2, st2 = _mm_layer(agg2.reshape(2, _NP, 128), h2,
                        degp, Wl2, bl2.reshape(1, _H), Wr2, "concat")
    node_emb, pooled, cnt = _norm3_layer(y2, st2, g2.reshape(1, _H),
                                         b2.reshape(1, _H), bt)

    logits, embedding = _mlp_head(
        pooled, cnt, radiomics, rg.reshape(1, 64), rb.reshape(1, 64),
        Wc1, bc1.reshape(1, 64), Wc2, bc2.reshape(1, 32),
        Wc3, bc3.reshape(1, 2), We, be.reshape(1, _H + 64))
    return (logits, embedding, node_emb)


# speedup vs baseline: 1.0470x; 1.0081x over previous
import functools

import jax
import jax.numpy as jnp
from jax import lax
from jax.experimental import pallas as pl
from jax.experimental.pallas import tpu as pltpu
import jax.experimental.pallas.tpu_sc as plsc

_N = 10000
_E = 320000
_B = 16
_H = 256
_NC = 2
_NS = 16
_K = 128
_NP = 10240
_RPT = _NP // _NS
_RS = 2560
_TRS = _RS // _NS
_RE = 1280
_TRE = _RE // _NS
_BR = 5000


def _zero_rows(buf, n):
    zero16 = jnp.zeros((16,), jnp.float32)

    def _z(i, _):
        for j in range(128 // 16):
            buf[i, pl.ds(j * 16, 16)] = zero16
        return _
    lax.fori_loop(0, n, _z, None)


def _zero_acc(acc, rows, s):
    r0 = s * _RPT
    for j in range(5):
        pltpu.sync_copy(rows.at[pl.ds(0, _K)],
                        acc.at[pl.ds(r0 + j * _K, _K)])


def _acc_writeout(acc, out, c, s):
    r0 = s * _RPT
    pltpu.sync_copy(acc.at[pl.ds(r0, _RPT)],
                    out.at[pl.ds(c * _NP + r0, _RPT)])


def _agg_pipeline(table, acc, srcs, dst, sbase, dbase, nchunks,
                  iss, ids, isems, rows, gsems, dhook=None, coff=None):
    ssems = gsems[2:]
    gsems = gsems[:2]

    def ld(chunk, j):
        off = chunk * _K
        pltpu.async_copy(srcs.at[pl.ds(sbase + off, _K)], iss[j], isems[j])
        pltpu.async_copy(dst.at[pl.ds(dbase + off, _K)], ids[j], isems[j])

    def wld(j):
        pltpu.make_async_copy(srcs.at[pl.ds(0, _K)], iss[j], isems[j]).wait()
        pltpu.make_async_copy(dst.at[pl.ds(0, _K)], ids[j], isems[j]).wait()

    def g(j, slot):
        if coff is not None:
            for k in range(8):
                iss[j][pl.ds(k * 16, 16)] = iss[j][pl.ds(k * 16, 16)] + coff
        pltpu.async_copy(table.at[iss[j]], rows[slot], gsems[slot])

    def wg(slot):
        pltpu.make_async_copy(table.at[pl.ds(0, _K)], rows[slot],
                              gsems[slot]).wait()

    def sc(j, slot):
        pltpu.async_copy(rows[slot], acc.at[ids[j]], ssems[slot], add=True)
        if dhook is not None:
            dhook(j)

    def wsc(j, slot):
        pltpu.make_async_copy(rows[slot], acc.at[ids[j]], ssems[slot]).wait()

    ld(0, 0)
    ld(1, 1)
    ld(2, 2)
    wld(0)
    g(0, 0)

    nb = nchunks // 4

    def _body(t, carry):
        b = 4 * t
        nxt = t < nb - 1
        ld(b + 3, 3)
        wld(1)

        @pl.when(t > 0)
        def _():
            wsc(3, 1)
        g(1, 1)
        wg(0)
        sc(0, 0)

        @pl.when(nxt)
        def _():
            ld(b + 4, 0)
        wld(2)
        wsc(0, 0)
        g(2, 0)
        wg(1)
        sc(1, 1)

        @pl.when(nxt)
        def _():
            ld(b + 5, 1)
        wld(3)
        wsc(1, 1)
        g(3, 1)
        wg(0)
        sc(2, 0)

        @pl.when(nxt)
        def _():
            ld(b + 6, 2)
            wld(0)
            wsc(2, 0)
            g(0, 0)
        wg(1)
        sc(3, 1)
        return carry
    lax.fori_loop(0, nb, _body, None)
    wsc(2, 0)
    wsc(3, 1)


def _sc_agg_common(table, srcs, dst, agg, sbase, dbase, nchunks, refs):
    (acc, i0, i1, i2, i3, d0, d1, d2, d3, r0, r1,
     m0, m1, m2, m3, gs0, gs1, ss0, ss1) = refs
    c = lax.axis_index("c")
    s = lax.axis_index("s")
    _zero_rows(r0, _K)
    _zero_acc(acc, r0, s)
    plsc.subcore_barrier()
    _agg_pipeline(table, acc, srcs, dst, sbase, dbase, nchunks,
                  (i0, i1, i2, i3), (d0, d1, d2, d3), (m0, m1, m2, m3),
                  (r0, r1), (gs0, gs1, ss0, ss1), coff=c * _N)
    plsc.subcore_barrier()
    _acc_writeout(acc, agg, c, s)


def _sc_slab_body(table, srcs, dst, agg, *refs):
    s = lax.axis_index("s")
    base = (s * _TRS) * _K
    _sc_agg_common(table, srcs, dst, agg, base, base, _TRS, refs)


def _sc_esplit_body(table, srcs, dst, agg, deg, *refs):
    (acc, dacc, i0, i1, i2, i3, d0, d1, d2, d3, r0, r1,
     m0, m1, m2, m3, gs0, gs1, ss0, ss1, ones, dbuf) = refs
    c = lax.axis_index("c")
    s = lax.axis_index("s")
    zero16 = jnp.zeros((16,), jnp.float32)
    _zero_rows(r0, _K)
    _zero_acc(acc, r0, s)
    r0b = s * _RPT
    for j in range(8):
        ones[pl.ds(j * 16, 16)] = zero16
    for j in range(5):
        pltpu.sync_copy(ones, dacc.at[pl.ds(r0b + j * _K, _K)])
    one16 = jnp.ones((16,), jnp.float32)
    for j in range(8):
        ones[pl.ds(j * 16, 16)] = one16
    plsc.subcore_barrier()

    base = (c * _RE + s * _TRE) * _K
    ids = (d0, d1, d2, d3)

    def dhook(j):
        pltpu.sync_copy(ones, dacc.at[ids[j]], add=True)

    _agg_pipeline(table, acc, srcs, dst, base, base, _TRE,
                  (i0, i1, i2, i3), ids, (m0, m1, m2, m3),
                  (r0, r1), (gs0, gs1, ss0, ss1), dhook)
    plsc.subcore_barrier()
    _acc_writeout(acc, agg, c, s)

    pltpu.sync_copy(dacc.at[pl.ds(r0b, _RPT)], dbuf)
    for blk in range(5):
        def _brow(g, carry):
            v16 = dbuf[pl.ds(blk * _K + g * 16, 16)]
            for l in range(16):
                row = jnp.full((16,), v16[l], jnp.float32)
                for j in range(8):
                    r0[g * 16 + l, pl.ds(j * 16, 16)] = row
            return carry
        lax.fori_loop(0, 8, _brow, None)
        pltpu.sync_copy(r0, deg.at[pl.ds(c * _NP + r0b + blk * _K, _K)])


def _sc_mesh():
    return plsc.VectorSubcoreMesh(core_axis_name="c", subcore_axis_name="s",
                                  num_cores=_NC, num_subcores=_NS)


def _agg_scratch():
    return (
        (pltpu.VMEM_SHARED((_NP, 128), jnp.float32),)
        + (pltpu.VMEM((_K,), jnp.int32),) * 8
        + (pltpu.VMEM((_K, 128), jnp.float32),) * 2
        + (pltpu.SemaphoreType.DMA,) * 8
    )


_SC_OUT = jax.ShapeDtypeStruct((_NC * _NP, 128), jnp.float32)


@functools.lru_cache(maxsize=None)
def _make_sc_slab():
    return pl.kernel(_sc_slab_body, out_type=_SC_OUT,
                     mesh=_sc_mesh(), scratch_types=_agg_scratch())


@functools.lru_cache(maxsize=None)
def _make_sc_esplit():
    scratch = (
        (pltpu.VMEM_SHARED((_NP, 128), jnp.float32),
         pltpu.VMEM_SHARED((_NP,), jnp.float32))
        + (pltpu.VMEM((_K,), jnp.int32),) * 8
        + (pltpu.VMEM((_K, 128), jnp.float32),) * 2
        + (pltpu.SemaphoreType.DMA,) * 8
        + (pltpu.VMEM((_K,), jnp.float32),
           pltpu.VMEM((_RPT,), jnp.float32))
    )
    return pl.kernel(_sc_esplit_body, out_type=(_SC_OUT, _SC_OUT),
                     mesh=_sc_mesh(), scratch_types=scratch)


def _mm_body(mode, agg_ref, h_ref, deg_ref, wl_ref, bl_ref, wr_ref,
             y_ref, st_ref):
    dn = (((1,), (1,)), ((), ()))
    if mode == "sum":
        aggc = agg_ref[0] + agg_ref[1]
        hc = h_ref[...]
    else:
        aggc = jnp.concatenate([agg_ref[0], agg_ref[1]], axis=1)
        hc = jnp.concatenate([h_ref[0], h_ref[1]], axis=1)
    yl = lax.dot_general(aggc, wl_ref[...], dn,
                         preferred_element_type=jnp.float32)
    d = deg_ref[0, :, 0:1] + deg_ref[1, :, 0:1]
    rdeg = 1.0 / jnp.maximum(d, 1.0)
    y = (yl * rdeg + bl_ref[...]
         + lax.dot_general(hc, wr_ref[...], dn,
                           preferred_element_type=jnp.float32))
    y_ref[...] = y
    s1 = jnp.sum(y, axis=0, keepdims=True)
    s2 = jnp.sum(y * y, axis=0, keepdims=True)
    blk = jnp.concatenate([s1, s2, jnp.zeros((6, _H), jnp.float32)], axis=0)

    @pl.when(pl.program_id(0) == 0)
    def _():
        st_ref[...] = blk

    @pl.when(pl.program_id(0) != 0)
    def _():
        st_ref[...] = st_ref[...] + blk


def _mm_layer(agg, h, deg, wl, bl, wr, mode):
    nb = _N // _BR
    din = wl.shape[1]
    if mode == "sum":
        h_spec = pl.BlockSpec((_BR, 128), lambda i: (i, 0))
    else:
        h_spec = pl.BlockSpec((2, _BR, 128), lambda i: (0, i, 0))
    return pl.pallas_call(
        functools.partial(_mm_body, mode),
        grid=(nb,),
        in_specs=[
            pl.BlockSpec((2, _BR, 128), lambda i: (0, i, 0)),
            h_spec,
            pl.BlockSpec((2, _BR, 128), lambda i: (0, i, 0)),
            pl.BlockSpec((_H, din), lambda i: (0, 0)),
            pl.BlockSpec((1, _H), lambda i: (0, 0)),
            pl.BlockSpec((_H, din), lambda i: (0, 0)),
        ],
        out_specs=[
            pl.BlockSpec((_BR, _H), lambda i: (i, 0)),
            pl.BlockSpec((8, _H), lambda i: (0, 0)),
        ],
        out_shape=[
            jax.ShapeDtypeStruct((_N, _H), jnp.float32),
            jax.ShapeDtypeStruct((8, _H), jnp.float32),
        ],
    )(agg, h, deg, wl, bl, wr)


def _norm_body(y_ref, st_ref, g_ref, b_ref, out_ref):
    mean = st_ref[0:1, :] / _N
    var = st_ref[1:2, :] / _N - mean * mean
    inv = lax.rsqrt(var + 1e-5)
    h = jnp.maximum((y_ref[...] - mean) * inv * g_ref[...] + b_ref[...], 0.0)
    out_ref[0] = h[:, 0:128]
    out_ref[1] = h[:, 128:256]


def _norm_layer(y, st, g, b):
    nb = _N // _BR
    return pl.pallas_call(
        _norm_body,
        grid=(nb,),
        in_specs=[
            pl.BlockSpec((_BR, _H), lambda i: (i, 0)),
            pl.BlockSpec((8, _H), lambda i: (0, 0)),
            pl.BlockSpec((1, _H), lambda i: (0, 0)),
            pl.BlockSpec((1, _H), lambda i: (0, 0)),
        ],
        out_specs=pl.BlockSpec((2, _BR, 128), lambda i: (0, i, 0)),
        out_shape=jax.ShapeDtypeStruct((2, _N, 128), jnp.float32),
    )(y, st, g, b)


def _norm3_body(y_ref, st_ref, g_ref, b_ref, bt_ref,
                node_ref, pooled_ref, cnt_ref):
    mean = st_ref[0:1, :] / _N
    var = st_ref[1:2, :] / _N - mean * mean
    inv = lax.rsqrt(var + 1e-5)
    h = jnp.maximum((y_ref[...] - mean) * inv * g_ref[...] + b_ref[...], 0.0)
    node_ref[...] = h
    bt = bt_ref[...]
    io = lax.broadcasted_iota(jnp.int32, (1, _B), 1)
    oh = (bt == io).astype(jnp.float32)
    dn0 = (((0,), (0,)), ((), ()))
    pc = lax.dot_general(oh, h, dn0, preferred_element_type=jnp.float32)
    cc = lax.dot_general(oh, jnp.ones((_BR, _B), jnp.float32), dn0,
                         preferred_element_type=jnp.float32)

    @pl.when(pl.program_id(0) == 0)
    def _():
        pooled_ref[...] = pc
        cnt_ref[...] = cc

    @pl.when(pl.program_id(0) != 0)
    def _():
        pooled_ref[...] = pooled_ref[...] + pc
        cnt_ref[...] = cnt_ref[...] + cc


def _norm3_layer(y, st, g, b, bt):
    nb = _N // _BR
    return pl.pallas_call(
        _norm3_body,
        grid=(nb,),
        in_specs=[
            pl.BlockSpec((_BR, _H), lambda i: (i, 0)),
            pl.BlockSpec((8, _H), lambda i: (0, 0)),
            pl.BlockSpec((1, _H), lambda i: (0, 0)),
            pl.BlockSpec((1, _H), lambda i: (0, 0)),
            pl.BlockSpec((_BR, 1), lambda i: (i, 0)),
        ],
        out_specs=[
            pl.BlockSpec((_BR, _H), lambda i: (i, 0)),
            pl.BlockSpec((_B, _H), lambda i: (0, 0)),
            pl.BlockSpec((_B, _B), lambda i: (0, 0)),
        ],
        out_shape=[
            jax.ShapeDtypeStruct((_N, _H), jnp.float32),
            jax.ShapeDtypeStruct((_B, _H), jnp.float32),
            jax.ShapeDtypeStruct((_B, _B), jnp.float32),
        ],
    )(y, st, g, b, bt)


def _mlp_body(pooled_ref, cnt_ref, rad_ref, rg_ref, rb_ref,
              wc1_ref, bc1_ref, wc2_ref, bc2_ref, wc3_ref, bc3_ref,
              we_ref, be_ref, logits_ref, emb_ref):
    dn = (((1,), (1,)), ((), ()))
    ge = pooled_ref[...] / jnp.maximum(cnt_ref[:, 0:1], 1.0)
    rad = rad_ref[...]
    m = jnp.mean(rad, axis=0, keepdims=True)
    v = jnp.mean(rad * rad, axis=0, keepdims=True) - m * m
    rn = (rad - m) * lax.rsqrt(v + 1e-5) * rg_ref[...] + rb_ref[...]
    fused = jnp.concatenate([ge, rn], axis=1)
    h1 = jnp.maximum(
        lax.dot_general(fused, wc1_ref[...], dn,
                        preferred_element_type=jnp.float32) + bc1_ref[...], 0.0)
    h2 = jnp.maximum(
        lax.dot_general(h1, wc2_ref[...], dn,
                        preferred_element_type=jnp.float32) + bc2_ref[...], 0.0)
    logits_ref[...] = (
        lax.dot_general(h2, wc3_ref[...], dn,
                        preferred_element_type=jnp.float32) + bc3_ref[...])
    emb_ref[...] = (
        lax.dot_general(fused, we_ref[...], dn,
                        preferred_element_type=jnp.float32) + be_ref[...])


def _mlp_head(pooled, cnt, rad, rg, rb, wc1, bc1, wc2, bc2, wc3, bc3, we, be):
    return pl.pallas_call(
        _mlp_body,
        out_shape=[
            jax.ShapeDtypeStruct((_B, 2), jnp.float32),
            jax.ShapeDtypeStruct((_B, _H + 64), jnp.float32),
        ],
    )(pooled, cnt, rad, rg, rb, wc1, bc1, wc2, bc2, wc3, bc3, we, be)


def kernel(x, edge_index, batch, radiomics,
           Wl0, bl0, Wr0, g0, b0, Wl1, bl1, Wr1, g1, b1,
           Wl2, bl2, Wr2, g2, b2, rg, rb,
           Wc1, bc1, Wc2, bc2, Wc3, bc3, We, be):
    src = edge_index[0]
    dst = edge_index[1]
    bt = batch.reshape(_N, 1)

    npad = _RS * _K - _E
    src_p = jnp.concatenate([src, jnp.arange(npad, dtype=jnp.int32) % _N])
    dst_p = jnp.concatenate(
        [dst, _N + jnp.arange(npad, dtype=jnp.int32) % (_NP - _N)])

    agg0r, degr = _make_sc_esplit()(x, src_p, dst_p)
    agg0 = agg0r.reshape(2, _NP, 128)
    degp = degr.reshape(2, _NP, 128)

    y0, st0 = _mm_layer(agg0, x, degp, Wl0, bl0.reshape(1, _H), Wr0, "sum")
    h1 = _norm_layer(y0, st0, g0.reshape(1, _H), b0.reshape(1, _H))

    agg1 = _make_sc_slab()(h1.reshape(2 * _N, 128), src_p, dst_p)
    y1, st1 = _mm_layer(agg1.reshape(2, _NP, 128), h1,
                        degp, Wl1, bl1.reshape(1, _H), Wr1, "concat")
    h2 = _norm_layer(y1, st1, g1.reshape(1, _H), b1.reshape(1, _H))

    agg2 = _make_sc_slab()(h2.reshape(2 * _N, 128), src_p, dst_p)
    y2, st2 = _mm_layer(agg2.reshape(2, _NP, 128), h2,
                        degp, Wl2, bl2.reshape(1, _H), Wr2, "concat")
    node_emb, pooled, cnt = _norm3_layer(y2, st2, g2.reshape(1, _H),
                                         b2.reshape(1, _H), bt)

    logits, embedding = _mlp_head(
        pooled, cnt, radiomics, rg.reshape(1, 64), rb.reshape(1, 64),
        Wc1, bc1.reshape(1, 64), Wc2, bc2.reshape(1, 32),
        Wc3, bc3.reshape(1, 2), We, be.reshape(1, _H + 64))
    return (logits, embedding, node_emb)
